# baseline (speedup 1.0000x reference)
import functools

import jax
import jax.numpy as jnp
from jax import lax
from jax.experimental import pallas as pl
from jax.experimental.pallas import tpu as pltpu
from jax.experimental.pallas import tpu_sc as plsc

_N_ATOMS = 10000
_N_EDGES = 320000
_NT = 4
_NR = 8
_P = 4
_CUT = 5.0
_WID = 0.5
_NW = 32
_EPW = _N_EDGES // _NW
_CH = 2000
_NCH = _EPW // _CH
_ACC = _N_ATOMS * 9
_ACCP = 98304
_PI = 3.141592653589793
_MAGIC = 0x5F3759DF


def _rsqrt(x):
    i = lax.bitcast_convert_type(x, jnp.int32)
    i = _MAGIC - lax.shift_right_logical(i, 1)
    y = lax.bitcast_convert_type(i, jnp.float32)
    for _ in range(2):
        y = y * (1.5 - 0.5 * x * y * y)
    return y


def _sin(p):
    u = p * p
    s = 1.0 / 362880.0
    s = s * u - 1.0 / 5040.0
    s = s * u + 1.0 / 120.0
    s = s * u - 1.0 / 6.0
    s = s * u + 1.0
    return p * s


def _cos(p):
    u = p * p
    s = -1.0 / 3628800.0
    s = s * u + 1.0 / 40320.0
    s = s * u - 1.0 / 720.0
    s = s * u + 1.0 / 24.0
    s = s * u - 0.5
    return s * u + 1.0


def _sc_partials(xs, ys, zs, cent, neig, spec, gtab):
    mesh = plsc.VectorSubcoreMesh(core_axis_name="c", subcore_axis_name="s")

    @functools.partial(
        pl.kernel,
        mesh=mesh,
        compiler_params=pltpu.CompilerParams(needs_layout_passes=False),
        out_type=jax.ShapeDtypeStruct((_NW, _ACCP), jnp.float32),
        scratch_types=[
            pltpu.VMEM((_N_ATOMS,), jnp.int32),
            pltpu.VMEM((_NT * _NT * _NR * 3,), jnp.float32),
            pltpu.VMEM((_ACCP,), jnp.float32),
            pltpu.VMEM((_CH,), jnp.float32),
            pltpu.VMEM((_CH,), jnp.float32),
            pltpu.VMEM((_CH,), jnp.float32),
            pltpu.VMEM((_CH,), jnp.int32),
            pltpu.VMEM((_CH,), jnp.int32),
            pltpu.VMEM((_CH,), jnp.float32),
            pltpu.VMEM((_CH,), jnp.float32),
            pltpu.VMEM((_CH,), jnp.float32),
            pltpu.VMEM((_CH,), jnp.int32),
            pltpu.VMEM((_CH,), jnp.int32),
            pltpu.SemaphoreType.DMA,
            pltpu.SemaphoreType.DMA,
        ],
    )
    def k(xs_h, ys_h, zs_h, c_h, n_h, sp_h, g_h, out_h,
          spec_v, g_v, acc_v,
          xb0, yb0, zb0, cb0, nb0, xb1, yb1, zb1, cb1, nb1,
          sem0, sem1):
        wid = lax.axis_index("s") * 2 + lax.axis_index("c")
        pltpu.sync_copy(sp_h, spec_v)
        pltpu.sync_copy(g_h, g_v)

        srcs = (xs_h, ys_h, zs_h, c_h, n_h)
        bufs = ((xb0, yb0, zb0, cb0, nb0), (xb1, yb1, zb1, cb1, nb1))
        sems = (sem0, sem1)
        base_w = wid * _EPW

        def issue(ch, slot):
            b0 = base_w + ch * _CH
            return [pltpu.async_copy(s.at[pl.ds(b0, _CH)], d, sems[slot])
                    for s, d in zip(srcs, bufs[slot])]

        pending = [issue(0, 0), None]

        zero16 = jnp.zeros((16,), jnp.float32)

        @plsc.parallel_loop(0, _ACCP // 96, 1, unroll=4)
        def zbody(i):
            for u in range(6):
                acc_v[pl.ds((i * 6 + u) * 16, 16)] = zero16

        for ch in range(_NCH):
            slot = ch & 1
            for d in pending[slot]:
                d.wait()
            if ch + 1 < _NCH:
                pending[1 - slot] = issue(ch + 1, 1 - slot)
            xb, yb, zb, cb, nb = bufs[slot]

            @plsc.parallel_loop(0, _CH // 16, 1, unroll=2)
            def body(i):
                o = i * 16
                x = xb[pl.ds(o, 16)]
                y = yb[pl.ds(o, 16)]
                z = zb[pl.ds(o, 16)]
                c = cb[pl.ds(o, 16)]
                n = nb[pl.ds(o, 16)]
                sc = plsc.load_gather(spec_v, [c])
                sn = plsc.load_gather(spec_v, [n])
                r2 = x * x + y * y + z * z + 1e-30
                rinv = _rsqrt(r2)
                r = r2 * rinv
                rc = jnp.minimum(r, _CUT)
                phi = rc * (_PI / _CUT) - (_PI / 2)
                s1 = _cos(phi)
                c2 = -2.0 * _sin(phi)
                pc = jnp.minimum(jnp.maximum((r - (_CUT - _WID)) * (_PI / _WID),
                                             0.0), _PI) - (_PI / 2)
                fc = 0.5 - 0.5 * _sin(pc)
                rad = [s1]
                sp_, s_ = s1, c2 * s1
                rad.append(s_)
                for _ in range(_NR - 2):
                    sp_, s_ = s_, c2 * s_ - sp_
                    rad.append(s_)
                gidx = sc * (_NT * _NR * 3) + sn * (_NR * 3)
                t = [None, None, None]
                for nr in range(_NR):
                    for b in range(3):
                        gv = plsc.load_gather(g_v, [gidx + (nr * 3 + b)])
                        term = rad[nr] * gv
                        t[b] = term if t[b] is None else t[b] + term
                w = (0.48860251190291992 * fc) * (rinv * rinv)
                tb = [t[0] * w, t[1] * w, t[2] * w]
                sh = (y, z, x)
                a9 = c * 9
                for m in range(3):
                    for b in range(3):
                        plsc.addupdate_scatter(acc_v, [a9 + (m * 3 + b)],
                                               sh[m] * tb[b])

        pltpu.sync_copy(acc_v, out_h.at[wid])

    return k(xs, ys, zs, cent, neig, spec, gtab)


def _sum_body(p_ref, o_ref):
    o_ref[...] = jnp.sum(p_ref[...], axis=0)


def kernel(interatomic_vectors, centers, neighbors, species, structures,
           atom_index_in_structure, W_alch, center_emb, W_contr):
    del structures, atom_index_in_structure
    v = interatomic_vectors
    xs = v[:, 0]
    ys = v[:, 1]
    zs = v[:, 2]
    cent = centers.astype(jnp.int32)
    neig = neighbors.astype(jnp.int32)
    spec = species.astype(jnp.int32)
    cw = (center_emb[:, :, None] * W_contr.T[None, :, :]).reshape(
        _NT, _NR, _P, 3)
    gtab = jnp.einsum('np,cqpb->cnqb', W_alch, cw).reshape(-1)
    gtab = gtab.astype(jnp.float32)

    partials = _sc_partials(xs, ys, zs, cent, neig, spec, gtab)
    nblk = 8
    out9 = pl.pallas_call(
        _sum_body,
        grid=(nblk,),
        in_specs=[pl.BlockSpec((_NW, _ACCP // nblk), lambda j: (0, j))],
        out_specs=pl.BlockSpec((_ACCP // nblk,), lambda j: (j,)),
        out_shape=jax.ShapeDtypeStruct((_ACCP,), jnp.float32),
    )(partials)
    return out9[:_ACC].reshape(_N_ATOMS, 3, 3)


# speedup vs baseline: 1.0242x; 1.0242x over previous
import functools

import jax
import jax.numpy as jnp
from jax import lax
from jax.experimental import pallas as pl
from jax.experimental.pallas import tpu as pltpu
from jax.experimental.pallas import tpu_sc as plsc

_N_ATOMS = 10000
_N_EDGES = 320000
_NT = 4
_NR = 8
_P = 4
_CUT = 5.0
_WID = 0.5
_NW = 32
_EPW = _N_EDGES // _NW
_CH = 2000
_NCH = _EPW // _CH
_ACC = _N_ATOMS * 9
_PI = 3.141592653589793
_MAGIC = 0x5F3759DF


def _rsqrt(x):
    i = lax.bitcast_convert_type(x, jnp.int32)
    i = _MAGIC - lax.shift_right_logical(i, 1)
    y = lax.bitcast_convert_type(i, jnp.float32)
    for _ in range(2):
        y = y * (1.5 - 0.5 * x * y * y)
    return y


def _sin(p):
    u = p * p
    s = 1.0 / 362880.0
    s = s * u - 1.0 / 5040.0
    s = s * u + 1.0 / 120.0
    s = s * u - 1.0 / 6.0
    s = s * u + 1.0
    return p * s


def _cos(p):
    u = p * p
    s = -1.0 / 3628800.0
    s = s * u + 1.0 / 40320.0
    s = s * u - 1.0 / 720.0
    s = s * u + 1.0 / 24.0
    s = s * u - 0.5
    return s * u + 1.0


def _sc_partials(xs, ys, zs, cent, neig, spec, gtab):
    mesh = plsc.VectorSubcoreMesh(core_axis_name="c", subcore_axis_name="s")

    @functools.partial(
        pl.kernel,
        mesh=mesh,
        compiler_params=pltpu.CompilerParams(needs_layout_passes=False),
        out_type=jax.ShapeDtypeStruct((_NW, _ACC), jnp.float32),
        scratch_types=[
            pltpu.VMEM((_N_ATOMS,), jnp.int32),
            pltpu.VMEM((_NT * _NT * _NR * 3,), jnp.float32),
            pltpu.VMEM((_ACC,), jnp.float32),
            pltpu.VMEM((_CH,), jnp.float32),
            pltpu.VMEM((_CH,), jnp.float32),
            pltpu.VMEM((_CH,), jnp.float32),
            pltpu.VMEM((_CH,), jnp.int32),
            pltpu.VMEM((_CH,), jnp.int32),
            pltpu.VMEM((_CH,), jnp.float32),
            pltpu.VMEM((_CH,), jnp.float32),
            pltpu.VMEM((_CH,), jnp.float32),
            pltpu.VMEM((_CH,), jnp.int32),
            pltpu.VMEM((_CH,), jnp.int32),
            pltpu.SemaphoreType.DMA,
            pltpu.SemaphoreType.DMA,
        ],
    )
    def k(xs_h, ys_h, zs_h, c_h, n_h, sp_h, g_h, out_h,
          spec_v, g_v, acc_v,
          xb0, yb0, zb0, cb0, nb0, xb1, yb1, zb1, cb1, nb1,
          sem0, sem1):
        wid = lax.axis_index("s") * 2 + lax.axis_index("c")
        pltpu.sync_copy(sp_h, spec_v)
        pltpu.sync_copy(g_h, g_v)

        srcs = (xs_h, ys_h, zs_h, c_h, n_h)
        bufs = ((xb0, yb0, zb0, cb0, nb0), (xb1, yb1, zb1, cb1, nb1))
        sems = (sem0, sem1)
        base_w = wid * _EPW

        def issue(ch, slot):
            b0 = base_w + ch * _CH
            return [pltpu.async_copy(s.at[pl.ds(b0, _CH)], d, sems[slot])
                    for s, d in zip(srcs, bufs[slot])]

        pending = [issue(0, 0), None]

        zero16 = jnp.zeros((16,), jnp.float32)

        @plsc.parallel_loop(0, _ACC // 80, 1, unroll=4)
        def zbody(i):
            for u in range(5):
                acc_v[pl.ds((i * 5 + u) * 16, 16)] = zero16

        for ch in range(_NCH):
            slot = ch & 1
            for d in pending[slot]:
                d.wait()
            if ch + 1 < _NCH:
                pending[1 - slot] = issue(ch + 1, 1 - slot)
            xb, yb, zb, cb, nb = bufs[slot]

            @plsc.parallel_loop(0, _CH // 16, 1, unroll=2)
            def body(i):
                o = i * 16
                x = xb[pl.ds(o, 16)]
                y = yb[pl.ds(o, 16)]
                z = zb[pl.ds(o, 16)]
                c = cb[pl.ds(o, 16)]
                n = nb[pl.ds(o, 16)]
                sc = plsc.load_gather(spec_v, [c])
                sn = plsc.load_gather(spec_v, [n])
                r2 = x * x + y * y + z * z + 1e-30
                rinv = _rsqrt(r2)
                r = r2 * rinv
                rc = jnp.minimum(r, _CUT)
                phi = rc * (_PI / _CUT) - (_PI / 2)
                s1 = _cos(phi)
                c2 = -2.0 * _sin(phi)
                pc = jnp.minimum(jnp.maximum((r - (_CUT - _WID)) * (_PI / _WID),
                                             0.0), _PI) - (_PI / 2)
                fc = 0.5 - 0.5 * _sin(pc)
                rad = [s1]
                sp_, s_ = s1, c2 * s1
                rad.append(s_)
                for _ in range(_NR - 2):
                    sp_, s_ = s_, c2 * s_ - sp_
                    rad.append(s_)
                gidx = sc * (_NT * _NR * 3) + sn * (_NR * 3)
                t = [None, None, None]
                for nr in range(_NR):
                    for b in range(3):
                        gv = plsc.load_gather(g_v, [gidx + (nr * 3 + b)])
                        term = rad[nr] * gv
                        t[b] = term if t[b] is None else t[b] + term
                w = (0.48860251190291992 * fc) * (rinv * rinv)
                tb = [t[0] * w, t[1] * w, t[2] * w]
                sh = (y, z, x)
                a9 = c * 9
                for m in range(3):
                    for b in range(3):
                        plsc.addupdate_scatter(acc_v, [a9 + (m * 3 + b)],
                                               sh[m] * tb[b])

        pltpu.sync_copy(acc_v, out_h.at[wid])

    return k(xs, ys, zs, cent, neig, spec, gtab)


def _sum_body(p_ref, o_ref):
    o_ref[...] = jnp.sum(p_ref[...], axis=0)


def kernel(interatomic_vectors, centers, neighbors, species, structures,
           atom_index_in_structure, W_alch, center_emb, W_contr):
    del structures, atom_index_in_structure
    v = interatomic_vectors
    xs = v[:, 0]
    ys = v[:, 1]
    zs = v[:, 2]
    cent = centers.astype(jnp.int32)
    neig = neighbors.astype(jnp.int32)
    spec = species.astype(jnp.int32)
    cw = (center_emb[:, :, None] * W_contr.T[None, :, :]).reshape(
        _NT, _NR, _P, 3)
    gtab = jnp.einsum('np,cqpb->cnqb', W_alch, cw).reshape(-1)
    gtab = gtab.astype(jnp.float32)

    partials = _sc_partials(xs, ys, zs, cent, neig, spec, gtab)
    out9 = pl.pallas_call(
        _sum_body,
        out_shape=jax.ShapeDtypeStruct((_ACC,), jnp.float32),
    )(partials)
    return out9.reshape(_N_ATOMS, 3, 3)
